# baseline (speedup 1.0000x reference)
import functools

import jax
import jax.numpy as jnp
import numpy as np
from jax import lax
from jax.experimental import pallas as pl
from jax.experimental.pallas import tpu as pltpu
from jax.experimental.pallas import tpu_sc as plsc

_VOCAB_SIZES = [100003, 100019, 100043, 100049, 100057, 100069, 100103, 100109]
_OFFSETS = [int(x) for x in np.cumsum([0] + _VOCAB_SIZES[:-1])]

_NUM_CORES = 2
_NUM_SUBCORES = 16
_NUM_WORKERS = _NUM_CORES * _NUM_SUBCORES
_LANES = 16
_CHUNK = 128
_GROUPS = 4
_H = 8


@functools.partial(jax.jit, static_argnames=("b", "s", "h", "d"))
def _mhe_lookup(ids, table, *, b, s, h, d):
    n = b * s * h
    n_per_w = n // _NUM_WORKERS
    s_per_w = n_per_w // h
    rows_per_w = n_per_w // _GROUPS
    chunks_per_group = rows_per_w // _CHUNK
    idx_rows = n_per_w // _CHUNK
    mesh = plsc.VectorSubcoreMesh(core_axis_name="c", subcore_axis_name="s")

    @functools.partial(
        pl.kernel,
        mesh=mesh,
        out_type=jax.ShapeDtypeStruct((n // _GROUPS, _GROUPS * d), jnp.float32),
        scratch_types=[
            pltpu.VMEM((idx_rows, _CHUNK), jnp.int32),
            pltpu.VMEM((idx_rows, _CHUNK), jnp.int32),
            pltpu.VMEM((rows_per_w, d), jnp.float32),
            pltpu.SemaphoreType.DMA,
        ],
        compiler_params=pltpu.CompilerParams(
            use_tc_tiling_on_sc=False, needs_layout_passes=False
        ),
    )
    def k(ids_hbm, table_hbm, out_hbm, idx8_v, idx_v, rows_v, sem):
        wid = lax.axis_index("s") * _NUM_CORES + lax.axis_index("c")

        pltpu.sync_copy(ids_hbm.at[pl.ds(wid * idx_rows, idx_rows)], idx8_v)

        iota = lax.iota(jnp.int32, _LANES)
        parity = iota & 1
        lane_pat = 8 * (iota >> 1) + _GROUPS * parity
        for j in range(_GROUPS):
            off_j = _OFFSETS[j] + (_OFFSETS[j + _GROUPS] - _OFFSETS[j]) * parity
            for c in range(chunks_per_group):
                row = j * chunks_per_group + c
                for t in range(_CHUNK // _LANES):
                    rows_i = jnp.full((_LANES,), 4 * c + t // 2, jnp.int32)
                    lanes_i = lane_pat + (64 * (t % 2) + j)
                    vals = plsc.load_gather(idx8_v, [rows_i, lanes_i]) + off_j
                    idx_v[row, pl.ds(16 * t, _LANES)] = vals

        out_base = wid * rows_per_w
        for j in range(_GROUPS):
            copies = []
            for c in range(chunks_per_group):
                row = j * chunks_per_group + c
                copies.append(
                    pltpu.async_copy(
                        table_hbm.at[idx_v.at[row]],
                        rows_v.at[pl.ds(c * _CHUNK, _CHUNK)],
                        sem,
                    )
                )
            for cp in copies:
                cp.wait()
            pltpu.sync_copy(
                rows_v,
                out_hbm.at[pl.ds(out_base, rows_per_w), pl.ds(j * d, d)],
            )

    return k(ids, table)


def kernel(input_ids, table):
    b, s, h = input_ids.shape
    d = table.shape[1]
    n = b * s * h
    ids2d = input_ids.reshape(n // 128, 128)
    out = _mhe_lookup(ids2d, table, b=b, s=s, h=h, d=d)
    one_f = table[0, 0] * 0.0 + 1.0
    return out.reshape(b, s, h, d) * one_f
